# baseline (speedup 1.0000x reference)
import jax
import jax.numpy as jnp
from jax.experimental import pallas as pl
from jax.experimental.pallas import tpu as pltpu


def _make_pos_kernel(b, d, h, w):
    n = h * w
    rows_per_ch = n // 128

    def _pos_kernel(row_ref, col_ref, out_ref, scratch_ref, sem):
        lane = jax.lax.broadcasted_iota(jnp.int32, (w, 128), 1)
        jrow = jax.lax.broadcasted_iota(jnp.int32, (w, 128), 0)
        sel_top = (lane % w == jrow).astype(jnp.float32)
        col4 = jax.lax.dot_general(
            col_ref[0:w, :], sel_top,
            dimension_numbers=(((0,), (0,)), ((), ())),
            preferred_element_type=jnp.float32)
        top = jnp.broadcast_to(col4[:, None, :], (d, rows_per_ch, 128))
        scratch_ref[0:d * rows_per_ch] = top.reshape(d * rows_per_ch, 128)
        for k in range(rows_per_ch):
            sel_k = ((lane // w) + (128 // w) * k == jrow).astype(jnp.float32)
            bot_k = jax.lax.dot_general(
                row_ref[0:h, :], sel_k,
                dimension_numbers=(((0,), (0,)), ((), ())),
                preferred_element_type=jnp.float32)
            scratch_ref[pl.Slice(d * rows_per_ch + k, d, rows_per_ch), :] = (
                bot_k)
        copies = [
            pltpu.make_async_copy(scratch_ref, out_ref.at[i], sem.at[i])
            for i in range(b)
        ]
        for c in copies:
            c.start()
        for c in copies:
            c.wait()

    return _pos_kernel


def kernel(tensor, row_embed, col_embed):
    b = tensor.shape[0]
    h, w = tensor.shape[-2], tensor.shape[-1]
    d = row_embed.shape[1]
    n = h * w

    out = pl.pallas_call(
        _make_pos_kernel(b, d, h, w),
        in_specs=[
            pl.BlockSpec(row_embed.shape, lambda: (0, 0)),
            pl.BlockSpec(col_embed.shape, lambda: (0, 0)),
        ],
        out_specs=pl.BlockSpec(memory_space=pl.ANY),
        out_shape=jax.ShapeDtypeStruct((b, 2 * d * n // 128, 128),
                                       jnp.float32),
        scratch_shapes=[
            pltpu.VMEM((2 * d * n // 128, 128), jnp.float32),
            pltpu.SemaphoreType.DMA((b,)),
        ],
    )(row_embed, col_embed)
    return out.reshape(b, 2 * d, h, w)


# speedup vs baseline: 13.3473x; 13.3473x over previous
import jax
import jax.numpy as jnp
from jax.experimental import pallas as pl
from jax.experimental.pallas import tpu as pltpu


def _make_pos_kernel(b, d, h, w):

    def _pos_kernel(row_ref, col_ref, out_ref, scratch_ref, sem):
        col = col_ref[0:w, :]
        row = row_ref[0:h, :]
        scratch_ref[:, :, 0:d] = jnp.broadcast_to(col[None, :, :], (h, w, d))
        scratch_ref[:, :, d:2 * d] = jnp.broadcast_to(
            row[:, None, :], (h, w, d))
        copies = [
            pltpu.make_async_copy(scratch_ref, out_ref.at[i], sem.at[i])
            for i in range(b)
        ]
        for c in copies:
            c.start()
        for c in copies:
            c.wait()

    return _pos_kernel


def kernel(tensor, row_embed, col_embed):
    b = tensor.shape[0]
    h, w = tensor.shape[-2], tensor.shape[-1]
    d = row_embed.shape[1]

    out = pl.pallas_call(
        _make_pos_kernel(b, d, h, w),
        in_specs=[
            pl.BlockSpec(row_embed.shape, lambda: (0, 0)),
            pl.BlockSpec(col_embed.shape, lambda: (0, 0)),
        ],
        out_specs=pl.BlockSpec(memory_space=pl.ANY),
        out_shape=jax.ShapeDtypeStruct((b, h, w, 2 * d), jnp.float32),
        scratch_shapes=[
            pltpu.VMEM((h, w, 2 * d), jnp.float32),
            pltpu.SemaphoreType.DMA((b,)),
        ],
    )(row_embed, col_embed)
    return out.transpose(0, 3, 1, 2)
